# baseline (speedup 1.0000x reference)
import functools

import jax
import jax.numpy as jnp
from jax import lax
from jax.experimental import pallas as pl
from jax.experimental.pallas import tpu as pltpu
from jax.experimental.pallas import tpu_sc as plsc

D = 128
NPAD = 10240
RPT = NPAD // 16
B = 128
NW = 32
CORE0_SHARE_PCT = 72


def _make_sc_counts(K):
    mesh = plsc.VectorSubcoreMesh(core_axis_name="c", subcore_axis_name="s")

    @functools.partial(
        pl.kernel,
        out_type=jax.ShapeDtypeStruct((2, NPAD, D), jnp.float32),
        mesh=mesh,
        scratch_types=[
            pltpu.VMEM((K, B), jnp.int32),
            pltpu.VMEM((B, D), jnp.float32),
            pltpu.VMEM((64, D), jnp.float32),
            pltpu.VMEM_SHARED((NPAD, D), jnp.float32),
        ],
    )
    def k(dst_hbm, cnt_hbm, didx, ones, zbuf, cnt_sh):
        c = lax.axis_index("c")
        s = lax.axis_index("s")
        wid = c * 16 + s

        zero16 = jnp.zeros((16,), jnp.float32)
        one0 = jnp.where(lax.iota(jnp.int32, 16) == 0,
                         jnp.float32(1.0), jnp.float32(0.0))

        def fill_z(i, _):
            for u in range(8):
                zbuf[i, pl.ds(u * 16, 16)] = zero16
            return 0
        lax.fori_loop(0, 64, fill_z, 0)

        def fill_ones(i, _):
            ones[i, pl.ds(0, 16)] = one0
            for u in range(1, 8):
                ones[i, pl.ds(u * 16, 16)] = zero16
            return 0
        lax.fori_loop(0, B, fill_ones, 0)

        for t in range(RPT // 64):
            pltpu.sync_copy(zbuf, cnt_sh.at[pl.ds(s * RPT + t * 64, 64)])
        plsc.subcore_barrier()

        pltpu.sync_copy(dst_hbm.at[wid], didx)

        def body(j, _):
            pltpu.sync_copy(ones, cnt_sh.at[didx.at[j]], add=True)
            return 0
        lax.fori_loop(0, K, body, 0)

        plsc.subcore_barrier()
        pltpu.sync_copy(cnt_sh.at[pl.ds(s * RPT, RPT)],
                        cnt_hbm.at[c, pl.ds(s * RPT, RPT)])

    return k


def _make_sc_aggregate(K, K0, K1):
    mesh = plsc.VectorSubcoreMesh(core_axis_name="c", subcore_axis_name="s")

    @functools.partial(
        pl.kernel,
        out_type=jax.ShapeDtypeStruct((2, NPAD, D), jnp.float32),
        mesh=mesh,
        scratch_types=[
            pltpu.VMEM((K, B), jnp.int32),
            pltpu.VMEM((K, B), jnp.int32),
            pltpu.VMEM((B, D), jnp.float32),
            pltpu.VMEM_SHARED((NPAD, D), jnp.float32),
            pltpu.SemaphoreType.DMA,
        ],
    )
    def k(x_hbm, src_hbm, dst_hbm, part_hbm,
          sidx, didx, rows, acc_sh, gsem):
        c = lax.axis_index("c")
        s = lax.axis_index("s")
        wid = c * 16 + s

        zero16 = jnp.zeros((16,), jnp.float32)

        def fill_z(i, _):
            for u in range(8):
                rows[i, pl.ds(u * 16, 16)] = zero16
            return 0
        lax.fori_loop(0, B, fill_z, 0)
        for t in range(RPT // B):
            pltpu.sync_copy(rows, acc_sh.at[pl.ds(s * RPT + t * B, B)])
        plsc.subcore_barrier()

        pltpu.sync_copy(src_hbm.at[wid], sidx)
        pltpu.sync_copy(dst_hbm.at[wid], didx)

        def body(j, _):
            pltpu.async_copy(x_hbm.at[sidx.at[j]], rows, gsem).wait()
            pltpu.sync_copy(rows, acc_sh.at[didx.at[j]], add=True)
            return 0

        kmin = min(K0, K1)
        lax.fori_loop(0, kmin, body, 0)

        @pl.when(c == (0 if K0 > K1 else 1))
        def _():
            lax.fori_loop(kmin, max(K0, K1), body, 0)

        plsc.subcore_barrier()
        pltpu.sync_copy(acc_sh.at[pl.ds(s * RPT, RPT)],
                        part_hbm.at[c, pl.ds(s * RPT, RPT)])

    return k


def _tc_dense(part, cnt, x, Wlt, Wrt, b):
    R = 512
    grid = (NPAD // R,)

    def body(p0_ref, p1_ref, c0_ref, c1_ref, x_ref, wl_ref, wr_ref, b_ref,
             o_ref):
        xs = x_ref[...]
        summed = p0_ref[0] + p1_ref[0] + xs
        inv = 1.0 / (c0_ref[0, :, 0:1] + c1_ref[0, :, 0:1] + 1.0)
        mean = summed * inv
        h = (jnp.dot(mean, wl_ref[...], preferred_element_type=jnp.float32)
             + jnp.dot(xs, wr_ref[...], preferred_element_type=jnp.float32)
             + b_ref[...])
        o_ref[...] = jnp.maximum(h, 0.0)

    rowblk = pl.BlockSpec((R, D), lambda i: (i, 0))
    slab0 = pl.BlockSpec((1, R, D), lambda i: (0, i, 0))
    slab1 = pl.BlockSpec((1, R, D), lambda i: (1, i, 0))
    full = pl.BlockSpec((D, D), lambda i: (0, 0))
    return pl.pallas_call(
        body,
        grid=grid,
        in_specs=[slab0, slab1, slab0, slab1, rowblk, full, full,
                  pl.BlockSpec((1, D), lambda i: (0, 0))],
        out_specs=rowblk,
        out_shape=jax.ShapeDtypeStruct((NPAD, D), jnp.float32),
    )(part, part, cnt, cnt, x, Wlt, Wrt, b)


def kernel(node_features, edge_index, W_l1, W_r1, b1, W_l2, W_r2, b2):
    n = node_features.shape[0]
    e = edge_index.shape[1]
    Kc = -(-e // (NW * B))
    epad = NW * Kc * B

    src = jnp.concatenate(
        [edge_index[0], jnp.zeros((epad - e,), jnp.int32)])
    dst = jnp.concatenate(
        [edge_index[1], jnp.full((epad - e,), NPAD - 1, jnp.int32)])
    dst3 = dst.reshape(NW, Kc, B)

    K0 = (2 * Kc * CORE0_SHARE_PCT + 50) // 100
    K1 = 2 * Kc - K0
    K = max(K0, K1)
    srcb = src.reshape(16 * (K0 + K1), B)
    dstb = dst.reshape(16 * (K0 + K1), B)

    def part(blk):
        a = blk[:16 * K0].reshape(16, K0, B)
        b = blk[16 * K0:].reshape(16, K1, B)
        a = jnp.pad(a, ((0, 0), (0, K - K0), (0, 0)))
        b = jnp.pad(b, ((0, 0), (0, K - K1), (0, 0)))
        return jnp.concatenate([a, b], axis=0)

    src3 = part(srcb)
    dst3a = part(dstb)

    x = jnp.pad(node_features, ((0, NPAD - n), (0, 0)))
    Wl1t = W_l1.T
    Wr1t = W_r1.T
    Wl2t = W_l2.T
    Wr2t = W_r2.T
    b1r = b1.reshape(1, D)
    b2r = b2.reshape(1, D)

    cnt = _make_sc_counts(Kc)(dst3)

    sc = _make_sc_aggregate(K, K0, K1)

    part1 = sc(x, src3, dst3a)
    h1 = _tc_dense(part1, cnt, x, Wl1t, Wr1t, b1r)

    part2 = sc(h1, src3, dst3a)
    h2 = _tc_dense(part2, cnt, h1, Wl2t, Wr2t, b2r)

    return h2[:n]


# speedup vs baseline: 1.0431x; 1.0431x over previous
import functools

import jax
import jax.numpy as jnp
from jax import lax
from jax.experimental import pallas as pl
from jax.experimental.pallas import tpu as pltpu
from jax.experimental.pallas import tpu_sc as plsc

D = 128
NPAD = 10240
RPT = NPAD // 16
B = 128
NW = 32
CORE0_SHARE_PCT = 72


def _make_sc_counts(K):
    mesh = plsc.VectorSubcoreMesh(core_axis_name="c", subcore_axis_name="s")

    @functools.partial(
        pl.kernel,
        out_type=jax.ShapeDtypeStruct((2, NPAD, D), jnp.float32),
        mesh=mesh,
        scratch_types=[
            pltpu.VMEM((K, B), jnp.int32),
            pltpu.VMEM((B, D), jnp.float32),
            pltpu.VMEM((64, D), jnp.float32),
            pltpu.VMEM_SHARED((NPAD, D), jnp.float32),
        ],
    )
    def k(dst_hbm, cnt_hbm, didx, ones, zbuf, cnt_sh):
        c = lax.axis_index("c")
        s = lax.axis_index("s")
        wid = c * 16 + s

        zero16 = jnp.zeros((16,), jnp.float32)
        one0 = jnp.where(lax.iota(jnp.int32, 16) == 0,
                         jnp.float32(1.0), jnp.float32(0.0))

        def fill_z(i, _):
            for u in range(8):
                zbuf[i, pl.ds(u * 16, 16)] = zero16
            return 0
        lax.fori_loop(0, 64, fill_z, 0)

        def fill_ones(i, _):
            ones[i, pl.ds(0, 16)] = one0
            for u in range(1, 8):
                ones[i, pl.ds(u * 16, 16)] = zero16
            return 0
        lax.fori_loop(0, B, fill_ones, 0)

        for t in range(RPT // 64):
            pltpu.sync_copy(zbuf, cnt_sh.at[pl.ds(s * RPT + t * 64, 64)])
        plsc.subcore_barrier()

        pltpu.sync_copy(dst_hbm.at[wid], didx)

        def body(j, _):
            pltpu.sync_copy(ones, cnt_sh.at[didx.at[j]], add=True)
            return 0
        lax.fori_loop(0, K, body, 0)

        plsc.subcore_barrier()
        pltpu.sync_copy(cnt_sh.at[pl.ds(s * RPT, RPT)],
                        cnt_hbm.at[c, pl.ds(s * RPT, RPT)])

    return k


def _make_sc_aggregate(K, K0, K1):
    mesh = plsc.VectorSubcoreMesh(core_axis_name="c", subcore_axis_name="s")

    @functools.partial(
        pl.kernel,
        out_type=jax.ShapeDtypeStruct((2, NPAD, D), jnp.float32),
        mesh=mesh,
        scratch_types=[
            pltpu.VMEM((K, B), jnp.int32),
            pltpu.VMEM((K, B), jnp.int32),
            pltpu.VMEM((B, D), jnp.float32),
            pltpu.VMEM_SHARED((NPAD, D), jnp.float32),
            pltpu.SemaphoreType.DMA,
        ],
    )
    def k(x_hbm, src_hbm, dst_hbm, part_hbm,
          sidx, didx, rows, acc_sh, gsem):
        c = lax.axis_index("c")
        s = lax.axis_index("s")
        wid = c * 16 + s

        zero16 = jnp.zeros((16,), jnp.float32)

        def fill_z(i, _):
            for u in range(8):
                rows[i, pl.ds(u * 16, 16)] = zero16
            return 0
        lax.fori_loop(0, B, fill_z, 0)
        for t in range(RPT // B):
            pltpu.sync_copy(rows, acc_sh.at[pl.ds(s * RPT + t * B, B)])
        plsc.subcore_barrier()

        pltpu.sync_copy(src_hbm.at[wid], sidx)
        pltpu.sync_copy(dst_hbm.at[wid], didx)

        def body(j, _):
            pltpu.async_copy(x_hbm.at[sidx.at[j]], rows, gsem).wait()
            pltpu.sync_copy(rows, acc_sh.at[didx.at[j]], add=True)
            return 0

        kmin = min(K0, K1)
        lax.fori_loop(0, kmin, body, 0)

        @pl.when(c == (0 if K0 > K1 else 1))
        def _():
            lax.fori_loop(kmin, max(K0, K1), body, 0)

        plsc.subcore_barrier()
        pltpu.sync_copy(acc_sh.at[pl.ds(s * RPT, RPT)],
                        part_hbm.at[c, pl.ds(s * RPT, RPT)])

    return k


def _tc_dense(p0, p1, c0, c1, x, Wlt, Wrt, b):
    R = 512
    grid = (NPAD // R,)

    def body(p0_ref, p1_ref, c0_ref, c1_ref, x_ref, wl_ref, wr_ref, b_ref,
             o_ref):
        xs = x_ref[...]
        summed = p0_ref[...] + p1_ref[...] + xs
        inv = 1.0 / (c0_ref[...] + c1_ref[...] + 1.0)
        mean = summed * inv
        h = (jnp.dot(mean, wl_ref[...], preferred_element_type=jnp.float32)
             + jnp.dot(xs, wr_ref[...], preferred_element_type=jnp.float32)
             + b_ref[...])
        o_ref[...] = jnp.maximum(h, 0.0)

    rowblk = pl.BlockSpec((R, D), lambda i: (i, 0))
    colvec = pl.BlockSpec((R, 1), lambda i: (i, 0))
    full = pl.BlockSpec((D, D), lambda i: (0, 0))
    return pl.pallas_call(
        body,
        grid=grid,
        in_specs=[rowblk, rowblk, colvec, colvec, rowblk, full, full,
                  pl.BlockSpec((1, D), lambda i: (0, 0))],
        out_specs=rowblk,
        out_shape=jax.ShapeDtypeStruct((NPAD, D), jnp.float32),
    )(p0, p1, c0, c1, x, Wlt, Wrt, b)


def kernel(node_features, edge_index, W_l1, W_r1, b1, W_l2, W_r2, b2):
    n = node_features.shape[0]
    e = edge_index.shape[1]
    Kc = -(-e // (NW * B))
    epad = NW * Kc * B

    src = jnp.concatenate(
        [edge_index[0], jnp.zeros((epad - e,), jnp.int32)])
    dst = jnp.concatenate(
        [edge_index[1], jnp.full((epad - e,), NPAD - 1, jnp.int32)])
    dst3 = dst.reshape(NW, Kc, B)

    K0 = (2 * Kc * CORE0_SHARE_PCT + 50) // 100
    K1 = 2 * Kc - K0
    K = max(K0, K1)
    srcb = src.reshape(16 * (K0 + K1), B)
    dstb = dst.reshape(16 * (K0 + K1), B)

    def part(blk):
        a = blk[:16 * K0].reshape(16, K0, B)
        b = blk[16 * K0:].reshape(16, K1, B)
        a = jnp.pad(a, ((0, 0), (0, K - K0), (0, 0)))
        b = jnp.pad(b, ((0, 0), (0, K - K1), (0, 0)))
        return jnp.concatenate([a, b], axis=0)

    src3 = part(srcb)
    dst3a = part(dstb)

    x = jnp.pad(node_features, ((0, NPAD - n), (0, 0)))
    Wl1t = W_l1.T
    Wr1t = W_r1.T
    Wl2t = W_l2.T
    Wr2t = W_r2.T
    b1r = b1.reshape(1, D)
    b2r = b2.reshape(1, D)

    cnt = _make_sc_counts(Kc)(dst3)
    c0 = lax.slice(cnt[0], (0, 0), (NPAD, 1))
    c1 = lax.slice(cnt[1], (0, 0), (NPAD, 1))

    sc = _make_sc_aggregate(K, K0, K1)

    part1 = sc(x, src3, dst3a)
    h1 = _tc_dense(part1[0], part1[1], c0, c1, x, Wl1t, Wr1t, b1r)

    part2 = sc(h1, src3, dst3a)
    h2 = _tc_dense(part2[0], part2[1], c0, c1, h1, Wl2t, Wr2t, b2r)

    return h2[:n]
